# Initial kernel scaffold; baseline (speedup 1.0000x reference)
import jax
import jax.numpy as jnp
from jax.experimental import pallas as pl


def kernel(entity_id, neighbor_ids, path_ids, edge_ids, W, Lw, Lb):
    raise NotImplementedError("write your pallas kernel here")


# speedup vs baseline: 1.5888x; 1.5888x over previous
import functools

import jax
import jax.numpy as jnp
from jax import lax
from jax.experimental import pallas as pl
from jax.experimental.pallas import tpu as pltpu
from jax.experimental.pallas import tpu_sc as plsc

L = 16
NCH = 8
DIM = 128
RPT = 32
PART = (NCH + 1) * L

_GROUPS = [(0, 10), (10, 2), (12, 1)]
_ENTITY_TILE = 13


def _sc_body(idx_hbm, w_hbm, head_hbm, gw_hbm, loss_hbm,
             idx_v, rows_v, part_v, pie_v, slab_v, part2_v, parte_v,
             slab2_v, slab3_v, head_v, gw_v, loss_v,
             spmem1, spmem2, spmem3, sem):
  tid = lax.axis_index("s")
  cid = lax.axis_index("c")
  zero = jnp.zeros((L,), jnp.float32)

  pltpu.sync_copy(idx_hbm.at[pl.ds(tid * RPT, RPT)], idx_v)
  pltpu.async_copy(w_hbm.at[idx_v], rows_v, sem).wait()

  def accum_store(n):
    def body(r, carry):
      ch = [rows_v[r, pl.ds(c * L, L)] for c in range(NCH)]
      s = tuple(carry[c] + ch[c] for c in range(NCH))
      q = carry[NCH]
      for c in range(NCH):
        q = q + ch[c] * ch[c]
      return s + (q,)

    carry = lax.fori_loop(0, n, body, (zero,) * (NCH + 1))
    for c in range(NCH + 1):
      part_v[pl.ds(c * L, L)] = carry[c]
    pltpu.sync_copy(part_v, spmem1.at[pl.ds(tid * PART, PART)])

  @pl.when(tid < 10)
  def _():
    accum_store(20)

  @pl.when(jnp.logical_and(tid >= 10, tid < 12))
  def _():
    accum_store(25)

  @pl.when(tid == 12)
  def _():
    accum_store(20)

  plsc.subcore_barrier()

  def rsqrt16(x):
    i = plsc.bitcast(x, jnp.int32)
    i = jnp.int32(0x5F3759DF) - lax.shift_right_logical(i, 1)
    y = plsc.bitcast(i, jnp.float32)
    for _ in range(3):
      y = y * (1.5 - 0.5 * x * y * y)
    return y

  def fetch_partials():
    pltpu.sync_copy(spmem1, slab_v)

  def compute_pie(lo, g):
    tot = []
    for c in range(NCH + 1):
      a = slab_v[pl.ds(lo * PART + c * L, L)]
      for s_ in range(lo + 1, lo + g):
        a = a + slab_v[pl.ds(s_ * PART + c * L, L)]
      tot.append(a)
    rinv = rsqrt16(jnp.broadcast_to(jnp.sum(tot[NCH]), (L,)))
    for c in range(NCH):
      pie_v[pl.ds(c * L, L)] = tot[c] * rinv

  def exp_dot_pie(r):
    d = zero
    for c in range(NCH):
      d = d + rows_v[r, pl.ds(c * L, L)] * pie_v[pl.ds(c * L, L)]
    return jnp.exp(jnp.broadcast_to(jnp.sum(d), (L,)))

  def expsum_store(n):
    acc = lax.fori_loop(0, n, lambda r, a: a + exp_dot_pie(r), zero)
    part2_v[...] = acc
    pltpu.sync_copy(part2_v, spmem2.at[pl.ds(tid * L, L)])

  @pl.when(tid < 10)
  def _():
    fetch_partials()
    compute_pie(0, 10)
    expsum_store(20)

  @pl.when(jnp.logical_and(tid >= 10, tid < 12))
  def _():
    fetch_partials()
    compute_pie(10, 2)
    expsum_store(25)

  @pl.when(tid == 12)
  def _():
    fetch_partials()
    compute_pie(12, 1)
    expsum_store(20)

  @pl.when(tid == _ENTITY_TILE)
  def _():
    fetch_partials()
    for li, (lo, g) in enumerate(_GROUPS):
      compute_pie(lo, g)
      parte_v[pl.ds(li * L, L)] = exp_dot_pie(0)
    pltpu.sync_copy(parte_v, spmem3)

  plsc.subcore_barrier()

  @pl.when(jnp.logical_and(tid == 0, cid == 0))
  def _():
    pltpu.sync_copy(spmem2, slab2_v)
    pltpu.sync_copy(spmem3, slab3_v)
    pltpu.sync_copy(head_hbm, head_v)
    dn = slab2_v[pl.ds(0, L)]
    for t in range(1, 10):
      dn = dn + slab2_v[pl.ds(t * L, L)]
    dp = slab2_v[pl.ds(10 * L, L)] + slab2_v[pl.ds(11 * L, L)]
    de = slab2_v[pl.ds(12 * L, L)]
    pn = slab3_v[pl.ds(0, L)] / dn
    pp = slab3_v[pl.ds(L, L)] / dp
    pe = slab3_v[pl.ds(2 * L, L)] / de
    lane = lax.iota(jnp.int32, L)
    one = zero + 1.0
    pvec = jnp.where(lane == 0, pn,
                     jnp.where(lane == 1, pp,
                               jnp.where(lane == 2, pe,
                                         jnp.where(lane == 3, one, zero))))
    gw = jnp.broadcast_to(jnp.sum(pvec * head_v[...]), (L,))
    gw_v[...] = gw
    loss_v[...] = 1.0 - gw
    pltpu.sync_copy(gw_v, gw_hbm)
    pltpu.sync_copy(loss_v, loss_hbm)


_sc_kernel = functools.partial(
    pl.kernel,
    out_type=(jax.ShapeDtypeStruct((L,), jnp.float32),
              jax.ShapeDtypeStruct((L,), jnp.float32)),
    mesh=plsc.VectorSubcoreMesh(core_axis_name="c", subcore_axis_name="s",
                                num_cores=2, num_subcores=16),
    scratch_types=[
        pltpu.VMEM((RPT,), jnp.int32),
        pltpu.VMEM((RPT, DIM), jnp.float32),
        pltpu.VMEM((PART,), jnp.float32),
        pltpu.VMEM((DIM,), jnp.float32),
        pltpu.VMEM((16 * PART,), jnp.float32),
        pltpu.VMEM((L,), jnp.float32),
        pltpu.VMEM((3 * L,), jnp.float32),
        pltpu.VMEM((16 * L,), jnp.float32),
        pltpu.VMEM((3 * L,), jnp.float32),
        pltpu.VMEM((L,), jnp.float32),
        pltpu.VMEM((L,), jnp.float32),
        pltpu.VMEM((L,), jnp.float32),
        pltpu.VMEM_SHARED((16 * PART,), jnp.float32),
        pltpu.VMEM_SHARED((16 * L,), jnp.float32),
        pltpu.VMEM_SHARED((3 * L,), jnp.float32),
        pltpu.SemaphoreType.DMA,
    ],
    compiler_params=pltpu.CompilerParams(needs_layout_passes=False),
)(_sc_body)


def kernel(entity_id, neighbor_ids, path_ids, edge_ids, W, Lw, Lb):
  rows_n = jnp.pad(neighbor_ids.astype(jnp.int32).reshape(10, 20),
                   ((0, 0), (0, RPT - 20)))
  rows_p = jnp.pad(path_ids.astype(jnp.int32).reshape(2, 25),
                   ((0, 0), (0, RPT - 25)))
  rows_e = jnp.pad(edge_ids.astype(jnp.int32).reshape(1, 20),
                   ((0, 0), (0, RPT - 20)))
  rows_s = jnp.pad(entity_id.astype(jnp.int32).reshape(1, 1),
                   ((0, 0), (0, RPT - 1)))
  idx_mat = jnp.concatenate(
      [rows_n, rows_p, rows_e, rows_s, jnp.zeros((2, RPT), jnp.int32)],
      axis=0).reshape(16 * RPT)
  head = jnp.concatenate(
      [Lw.astype(jnp.float32).reshape(3), Lb.astype(jnp.float32).reshape(1),
       jnp.zeros((L - 4,), jnp.float32)])
  gw16, loss16 = _sc_kernel(idx_mat, W.astype(jnp.float32), head)
  return (gw16[0:1], loss16[0:1])


# speedup vs baseline: 2.0851x; 1.3123x over previous
import functools

import jax
import jax.numpy as jnp
from jax import lax
from jax.experimental import pallas as pl
from jax.experimental.pallas import tpu as pltpu
from jax.experimental.pallas import tpu_sc as plsc

L = 16
NCH = 8
DIM = 128
RPT = 32
PART = (NCH + 1) * L

_GROUPS = [(0, 10), (10, 2), (12, 1)]
_ENTITY_TILE = 13


def _sc_body(idx_hbm, w_hbm, head_hbm, gw_hbm, loss_hbm,
             idx_v, rows_v, part_v, pie_v, slab_v, part2_v, parte_v,
             slab2_v, slab3_v, head_v, gw_v, loss_v,
             spmem1, spmem2, spmem3, sem):
  tid = lax.axis_index("s")
  cid = lax.axis_index("c")
  zero = jnp.zeros((L,), jnp.float32)

  pltpu.sync_copy(idx_hbm.at[pl.ds(tid * RPT, RPT)], idx_v)
  pltpu.async_copy(w_hbm.at[idx_v], rows_v, sem).wait()

  def accum_store(n):
    def body(r, carry):
      ch = [rows_v[r, pl.ds(c * L, L)] for c in range(NCH)]
      s = tuple(carry[c] + ch[c] for c in range(NCH))
      q = carry[NCH]
      for c in range(NCH):
        q = q + ch[c] * ch[c]
      return s + (q,)

    carry = lax.fori_loop(0, n, body, (zero,) * (NCH + 1))
    for c in range(NCH + 1):
      part_v[pl.ds(c * L, L)] = carry[c]
    pltpu.sync_copy(part_v, spmem1.at[pl.ds(tid * PART, PART)])

  @pl.when(tid < 10)
  def _():
    accum_store(20)

  @pl.when(jnp.logical_and(tid >= 10, tid < 12))
  def _():
    accum_store(25)

  @pl.when(tid == 12)
  def _():
    accum_store(20)

  plsc.subcore_barrier()

  def rsqrt16(x):
    i = plsc.bitcast(x, jnp.int32)
    i = jnp.int32(0x5F3759DF) - lax.shift_right_logical(i, 1)
    y = plsc.bitcast(i, jnp.float32)
    for _ in range(3):
      y = y * (1.5 - 0.5 * x * y * y)
    return y

  def fetch_partials():
    pltpu.sync_copy(spmem1, slab_v)

  def compute_pie(lo, g):
    tot = []
    for c in range(NCH + 1):
      a = slab_v[pl.ds(lo * PART + c * L, L)]
      for s_ in range(lo + 1, lo + g):
        a = a + slab_v[pl.ds(s_ * PART + c * L, L)]
      tot.append(a)
    rinv = rsqrt16(jnp.broadcast_to(jnp.sum(tot[NCH]), (L,)))
    for c in range(NCH):
      pie_v[pl.ds(c * L, L)] = tot[c] * rinv

  def exp_dot_pie(r):
    d = zero
    for c in range(NCH):
      d = d + rows_v[r, pl.ds(c * L, L)] * pie_v[pl.ds(c * L, L)]
    return jnp.exp(jnp.broadcast_to(jnp.sum(d), (L,)))

  def expsum_store(n):
    acc = lax.fori_loop(0, n, lambda r, a: a + exp_dot_pie(r), zero)
    part2_v[...] = acc
    pltpu.sync_copy(part2_v, spmem2.at[pl.ds(tid * L, L)])

  @pl.when(tid < 10)
  def _():
    fetch_partials()
    compute_pie(0, 10)
    expsum_store(20)

  @pl.when(jnp.logical_and(tid >= 10, tid < 12))
  def _():
    fetch_partials()
    compute_pie(10, 2)
    expsum_store(25)

  @pl.when(tid == 12)
  def _():
    fetch_partials()
    compute_pie(12, 1)
    expsum_store(20)

  @pl.when(tid == _ENTITY_TILE)
  def _():
    fetch_partials()
    for li, (lo, g) in enumerate(_GROUPS):
      compute_pie(lo, g)
      parte_v[pl.ds(li * L, L)] = exp_dot_pie(0)
    pltpu.sync_copy(parte_v, spmem3)

  plsc.subcore_barrier()

  @pl.when(jnp.logical_and(tid == 0, cid == 0))
  def _():
    pltpu.sync_copy(spmem2, slab2_v)
    pltpu.sync_copy(spmem3, slab3_v)
    pltpu.sync_copy(head_hbm, head_v)
    dn = slab2_v[pl.ds(0, L)]
    for t in range(1, 10):
      dn = dn + slab2_v[pl.ds(t * L, L)]
    dp = slab2_v[pl.ds(10 * L, L)] + slab2_v[pl.ds(11 * L, L)]
    de = slab2_v[pl.ds(12 * L, L)]
    pn = slab3_v[pl.ds(0, L)] / dn
    pp = slab3_v[pl.ds(L, L)] / dp
    pe = slab3_v[pl.ds(2 * L, L)] / de
    lane = lax.iota(jnp.int32, L)
    one = zero + 1.0
    pvec = jnp.where(lane == 0, pn,
                     jnp.where(lane == 1, pp,
                               jnp.where(lane == 2, pe,
                                         jnp.where(lane == 3, one, zero))))
    gw = jnp.broadcast_to(jnp.sum(pvec * head_v[...]), (L,))
    gw_v[...] = gw
    loss_v[...] = 1.0 - gw
    pltpu.sync_copy(gw_v, gw_hbm)
    pltpu.sync_copy(loss_v, loss_hbm)


_sc_kernel = functools.partial(
    pl.kernel,
    out_type=(jax.ShapeDtypeStruct((L,), jnp.float32),
              jax.ShapeDtypeStruct((L,), jnp.float32)),
    mesh=plsc.VectorSubcoreMesh(core_axis_name="c", subcore_axis_name="s",
                                num_cores=1, num_subcores=16),
    scratch_types=[
        pltpu.VMEM((RPT,), jnp.int32),
        pltpu.VMEM((RPT, DIM), jnp.float32),
        pltpu.VMEM((PART,), jnp.float32),
        pltpu.VMEM((DIM,), jnp.float32),
        pltpu.VMEM((16 * PART,), jnp.float32),
        pltpu.VMEM((L,), jnp.float32),
        pltpu.VMEM((3 * L,), jnp.float32),
        pltpu.VMEM((16 * L,), jnp.float32),
        pltpu.VMEM((3 * L,), jnp.float32),
        pltpu.VMEM((L,), jnp.float32),
        pltpu.VMEM((L,), jnp.float32),
        pltpu.VMEM((L,), jnp.float32),
        pltpu.VMEM_SHARED((16 * PART,), jnp.float32),
        pltpu.VMEM_SHARED((16 * L,), jnp.float32),
        pltpu.VMEM_SHARED((3 * L,), jnp.float32),
        pltpu.SemaphoreType.DMA,
    ],
    compiler_params=pltpu.CompilerParams(needs_layout_passes=False),
)(_sc_body)


def kernel(entity_id, neighbor_ids, path_ids, edge_ids, W, Lw, Lb):
  rows_n = jnp.pad(neighbor_ids.astype(jnp.int32).reshape(10, 20),
                   ((0, 0), (0, RPT - 20)))
  rows_p = jnp.pad(path_ids.astype(jnp.int32).reshape(2, 25),
                   ((0, 0), (0, RPT - 25)))
  rows_e = jnp.pad(edge_ids.astype(jnp.int32).reshape(1, 20),
                   ((0, 0), (0, RPT - 20)))
  rows_s = jnp.pad(entity_id.astype(jnp.int32).reshape(1, 1),
                   ((0, 0), (0, RPT - 1)))
  idx_mat = jnp.concatenate(
      [rows_n, rows_p, rows_e, rows_s, jnp.zeros((2, RPT), jnp.int32)],
      axis=0).reshape(16 * RPT)
  head = jnp.concatenate(
      [Lw.astype(jnp.float32).reshape(3), Lb.astype(jnp.float32).reshape(1),
       jnp.zeros((L - 4,), jnp.float32)])
  gw16, loss16 = _sc_kernel(idx_mat, W.astype(jnp.float32), head)
  return (gw16[0:1], loss16[0:1])


# speedup vs baseline: 2.2693x; 1.0883x over previous
import functools

import jax
import jax.numpy as jnp
from jax import lax
from jax.experimental import pallas as pl
from jax.experimental.pallas import tpu as pltpu
from jax.experimental.pallas import tpu_sc as plsc

L = 16
NCH = 8
DIM = 128
RPT = 24
PART = (NCH + 1) * L

_GROUPS = [(0, 10), (10, 3), (13, 1)]
_ENTITY_TILE = 14
_NUMER_OFF = 16 * L


def _sc_body(idx_hbm, w_hbm, head_hbm, out_hbm,
             idx_v, rows_v, part_v, slab_v, part2_v, parte_v,
             slab2_v, head_v, out_v, spmem1, spmem2, sem):
  tid = lax.axis_index("s")
  cid = lax.axis_index("c")
  zero = jnp.zeros((L,), jnp.float32)
  is0 = jnp.logical_and(tid == 0, cid == 0)

  @pl.when(is0)
  def _():
    pltpu.sync_copy(head_hbm, head_v)

  pltpu.sync_copy(idx_hbm.at[pl.ds(tid * RPT, RPT)], idx_v)
  pltpu.async_copy(w_hbm.at[idx_v], rows_v, sem).wait()

  def accum_store(n):
    def body(r, carry):
      ch = [rows_v[r, pl.ds(c * L, L)] for c in range(NCH)]
      s = tuple(carry[c] + ch[c] for c in range(NCH))
      q = carry[NCH]
      for c in range(NCH):
        q = q + ch[c] * ch[c]
      return s + (q,)

    carry = lax.fori_loop(0, n, body, (zero,) * (NCH + 1))
    for c in range(NCH + 1):
      part_v[pl.ds(c * L, L)] = carry[c]
    pltpu.sync_copy(part_v, spmem1.at[pl.ds(tid * PART, PART)])

  @pl.when(tid < 10)
  def _():
    accum_store(20)

  @pl.when(jnp.logical_and(tid >= 10, tid < 12))
  def _():
    accum_store(17)

  @pl.when(tid == 12)
  def _():
    accum_store(16)

  @pl.when(tid == 13)
  def _():
    accum_store(20)

  plsc.subcore_barrier()

  def rsqrt16(x):
    i = plsc.bitcast(x, jnp.int32)
    i = jnp.int32(0x5F3759DF) - lax.shift_right_logical(i, 1)
    y = plsc.bitcast(i, jnp.float32)
    for _ in range(3):
      y = y * (1.5 - 0.5 * x * y * y)
    return y

  def fetch_partials():
    pltpu.sync_copy(spmem1, slab_v)

  def compute_pie(lo, g):
    tot = []
    for c in range(NCH + 1):
      a = slab_v[pl.ds(lo * PART + c * L, L)]
      for s_ in range(lo + 1, lo + g):
        a = a + slab_v[pl.ds(s_ * PART + c * L, L)]
      tot.append(a)
    rinv = rsqrt16(jnp.broadcast_to(jnp.sum(tot[NCH]), (L,)))
    return tuple(tot[c] * rinv for c in range(NCH))

  def exp_dot_pie(r, pie):
    d = rows_v[r, pl.ds(0, L)] * pie[0]
    for c in range(1, NCH):
      d = d + rows_v[r, pl.ds(c * L, L)] * pie[c]
    return jnp.exp(jnp.broadcast_to(jnp.sum(d), (L,)))

  def expsum_store(n, pie):
    acc = lax.fori_loop(0, n, lambda r, a: a + exp_dot_pie(r, pie), zero)
    part2_v[...] = acc
    pltpu.sync_copy(part2_v, spmem2.at[pl.ds(tid * L, L)])

  @pl.when(tid < 10)
  def _():
    fetch_partials()
    expsum_store(20, compute_pie(0, 10))

  @pl.when(jnp.logical_and(tid >= 10, tid < 12))
  def _():
    fetch_partials()
    expsum_store(17, compute_pie(10, 3))

  @pl.when(tid == 12)
  def _():
    fetch_partials()
    expsum_store(16, compute_pie(10, 3))

  @pl.when(tid == 13)
  def _():
    fetch_partials()
    expsum_store(20, compute_pie(13, 1))

  @pl.when(tid == _ENTITY_TILE)
  def _():
    fetch_partials()
    for li, (lo, g) in enumerate(_GROUPS):
      parte_v[pl.ds(li * L, L)] = exp_dot_pie(0, compute_pie(lo, g))
    pltpu.sync_copy(parte_v, spmem2.at[pl.ds(_NUMER_OFF, 3 * L)])

  plsc.subcore_barrier()

  @pl.when(is0)
  def _():
    pltpu.sync_copy(spmem2, slab2_v)
    dn = slab2_v[pl.ds(0, L)]
    for t in range(1, 10):
      dn = dn + slab2_v[pl.ds(t * L, L)]
    dp = (slab2_v[pl.ds(10 * L, L)] + slab2_v[pl.ds(11 * L, L)]
          + slab2_v[pl.ds(12 * L, L)])
    de = slab2_v[pl.ds(13 * L, L)]
    pn = slab2_v[pl.ds(_NUMER_OFF, L)] / dn
    pp = slab2_v[pl.ds(_NUMER_OFF + L, L)] / dp
    pe = slab2_v[pl.ds(_NUMER_OFF + 2 * L, L)] / de
    lane = lax.iota(jnp.int32, L)
    one = zero + 1.0
    pvec = jnp.where(lane == 0, pn,
                     jnp.where(lane == 1, pp,
                               jnp.where(lane == 2, pe,
                                         jnp.where(lane == 3, one, zero))))
    gw = jnp.broadcast_to(jnp.sum(pvec * head_v[...]), (L,))
    out_v[pl.ds(0, L)] = gw
    out_v[pl.ds(L, L)] = 1.0 - gw
    pltpu.sync_copy(out_v, out_hbm)


_sc_kernel = functools.partial(
    pl.kernel,
    out_type=(jax.ShapeDtypeStruct((2 * L,), jnp.float32),),
    mesh=plsc.VectorSubcoreMesh(core_axis_name="c", subcore_axis_name="s",
                                num_cores=1, num_subcores=16),
    scratch_types=[
        pltpu.VMEM((RPT,), jnp.int32),
        pltpu.VMEM((RPT, DIM), jnp.float32),
        pltpu.VMEM((PART,), jnp.float32),
        pltpu.VMEM((16 * PART,), jnp.float32),
        pltpu.VMEM((L,), jnp.float32),
        pltpu.VMEM((3 * L,), jnp.float32),
        pltpu.VMEM((19 * L,), jnp.float32),
        pltpu.VMEM((L,), jnp.float32),
        pltpu.VMEM((2 * L,), jnp.float32),
        pltpu.VMEM_SHARED((16 * PART,), jnp.float32),
        pltpu.VMEM_SHARED((19 * L,), jnp.float32),
        pltpu.SemaphoreType.DMA,
    ],
    compiler_params=pltpu.CompilerParams(needs_layout_passes=False),
)(_sc_body)


def kernel(entity_id, neighbor_ids, path_ids, edge_ids, W, Lw, Lb):
  n = neighbor_ids.astype(jnp.int32)
  p = path_ids.astype(jnp.int32)
  e = edge_ids.astype(jnp.int32)
  s = entity_id.astype(jnp.int32)
  rows_n = jnp.pad(n.reshape(10, 20), ((0, 0), (0, RPT - 20)))
  rows_p = jnp.stack([
      jnp.pad(p[0:17], (0, RPT - 17)),
      jnp.pad(p[17:34], (0, RPT - 17)),
      jnp.pad(p[34:50], (0, RPT - 16)),
  ])
  rows_e = jnp.pad(e.reshape(1, 20), ((0, 0), (0, RPT - 20)))
  rows_s = jnp.pad(s.reshape(1, 1), ((0, 0), (0, RPT - 1)))
  idx_mat = jnp.concatenate(
      [rows_n, rows_p, rows_e, rows_s, jnp.zeros((1, RPT), jnp.int32)],
      axis=0).reshape(16 * RPT)
  head = jnp.concatenate(
      [Lw.astype(jnp.float32).reshape(3), Lb.astype(jnp.float32).reshape(1),
       jnp.zeros((L - 4,), jnp.float32)])
  out, = _sc_kernel(idx_mat, W.astype(jnp.float32), head)
  return (out[0:1], out[L:L + 1])
